# Initial kernel scaffold; baseline (speedup 1.0000x reference)
import jax
import jax.numpy as jnp
from jax.experimental import pallas as pl


def kernel(x, embed):
    raise NotImplementedError("write your pallas kernel here")


# speedup vs baseline: 1.3698x; 1.3698x over previous
import functools

import jax
import jax.numpy as jnp
from jax import lax
from jax.experimental import pallas as pl
from jax.experimental.pallas import tpu as pltpu
from jax.experimental.pallas import tpu_sc as plsc


def _gather_kernel(n_total, n_chunks, chunk, d, idx_hbm, table_hbm, out_hbm,
                   idx_v, rows_v, sem):
    num_cores = 2
    wid = lax.axis_index("s") * num_cores + lax.axis_index("c")
    per_w = n_total // 32
    base = wid * per_w

    def body(i, _):
        off = base + i * chunk
        pltpu.sync_copy(idx_hbm.at[pl.ds(off, chunk)], idx_v)
        pltpu.async_copy(table_hbm.at[idx_v], rows_v, sem).wait()
        pltpu.sync_copy(rows_v, out_hbm.at[pl.ds(off, chunk)])
        return 0

    lax.fori_loop(0, n_chunks, body, 0)


def kernel(x, embed):
    b, s = x.shape
    v, d = embed.shape
    n = b * s
    chunk = 1024
    per_w = n // 32
    n_chunks = per_w // chunk

    xf = x.reshape(n).astype(jnp.int32)
    mesh = plsc.VectorSubcoreMesh(core_axis_name="c", subcore_axis_name="s")

    run = pl.kernel(
        functools.partial(_gather_kernel, n, n_chunks, chunk, d),
        mesh=mesh,
        out_type=jax.ShapeDtypeStruct((n, d), jnp.float32),
        scratch_types=[
            pltpu.VMEM((chunk,), jnp.int32),
            pltpu.VMEM((chunk, d), jnp.float32),
            pltpu.SemaphoreType.DMA,
        ],
        compiler_params=pltpu.CompilerParams(use_tc_tiling_on_sc=False),
    )
    out = run(xf, embed)
    return out.reshape(b, s, d)


# speedup vs baseline: 1.4071x; 1.0272x over previous
import functools

import jax
import jax.numpy as jnp
from jax import lax
from jax.experimental import pallas as pl
from jax.experimental.pallas import tpu as pltpu
from jax.experimental.pallas import tpu_sc as plsc

_NUM_WORKERS = 32
_CHUNK = 1280


def _gather_kernel(n_total, idx_hbm, table_hbm, out_hbm,
                   idx_v, buf0, buf1, semg0, semg1, semw0, semw1):
    wid = lax.axis_index("s") * 2 + lax.axis_index("c")
    per_w = n_total // _NUM_WORKERS
    n_chunks = per_w // _CHUNK
    n_pairs = n_chunks // 2
    base = wid * per_w
    c = _CHUNK

    pltpu.sync_copy(idx_hbm.at[pl.ds(base, per_w)], idx_v)

    def g_desc(i, buf, sem):
        return pltpu.make_async_copy(table_hbm.at[idx_v.at[pl.ds(i * c, c)]],
                                     buf, sem)

    def w_desc(i, buf, sem):
        return pltpu.make_async_copy(buf, out_hbm.at[pl.ds(base + i * c, c)],
                                     sem)

    g_desc(0, buf0, semg0).start()

    def body(j, _):
        i0 = 2 * j
        @pl.when(j > 0)
        def _wait_w1():
            w_desc(i0 - 1, buf1, semw1).wait()

        g_desc(i0 + 1, buf1, semg1).start()
        g_desc(i0, buf0, semg0).wait()
        w_desc(i0, buf0, semw0).start()
        g_desc(i0 + 1, buf1, semg1).wait()

        @pl.when(j < n_pairs - 1)
        def _next_g0():
            w_desc(i0, buf0, semw0).wait()
            g_desc(i0 + 2, buf0, semg0).start()

        w_desc(i0 + 1, buf1, semw1).start()

        @pl.when(j == n_pairs - 1)
        def _final_waits():
            w_desc(i0, buf0, semw0).wait()
            w_desc(i0 + 1, buf1, semw1).wait()

        return 0

    lax.fori_loop(0, n_pairs, body, 0)


def kernel(x, embed):
    b, s = x.shape
    v, d = embed.shape
    n = b * s

    xf = x.reshape(n).astype(jnp.int32)
    mesh = plsc.VectorSubcoreMesh(core_axis_name="c", subcore_axis_name="s")

    run = pl.kernel(
        functools.partial(_gather_kernel, n),
        mesh=mesh,
        out_type=jax.ShapeDtypeStruct((n, d), jnp.float32),
        scratch_types=[
            pltpu.VMEM((n // _NUM_WORKERS,), jnp.int32),
            pltpu.VMEM((_CHUNK, d), jnp.float32),
            pltpu.VMEM((_CHUNK, d), jnp.float32),
            pltpu.SemaphoreType.DMA,
            pltpu.SemaphoreType.DMA,
            pltpu.SemaphoreType.DMA,
            pltpu.SemaphoreType.DMA,
        ],
        compiler_params=pltpu.CompilerParams(use_tc_tiling_on_sc=False),
    )
    out = run(xf, embed)
    return out.reshape(b, s, d)
